# Initial kernel scaffold; baseline (speedup 1.0000x reference)
import jax
import jax.numpy as jnp
from jax.experimental import pallas as pl


def kernel(x, edge_index, conv_w, conv_b, W_l1, b_l1, W_r1, b_r1, att1, bias1, W_l2, b_l2, W_r2, b_r2, att2, bias2):
    raise NotImplementedError("write your pallas kernel here")


# speedup vs baseline: 8.7345x; 8.7345x over previous
import functools

import jax
import jax.numpy as jnp
from jax import lax
from jax.experimental import pallas as pl
from jax.experimental.pallas import tpu as pltpu
from jax.experimental.pallas import tpu_sc as plsc

N = 10000
NP = 10240
E_RAW = 320000
E_PAD = 331776
B = 64
BN = 1024
STRC = (NP // 16) // B


def _proj1_body(x48, wbig, cb, wl, bl, wr, br, xl, xr):
    xf = jnp.dot(x48[...], wbig[...], preferred_element_type=jnp.float32, precision=lax.Precision.HIGHEST)
    xf = xf + cb[...]
    xl[...] = jnp.dot(xf, wl[...], preferred_element_type=jnp.float32, precision=lax.Precision.HIGHEST) + bl[...]
    xr[...] = jnp.dot(xf, wr[...], preferred_element_type=jnp.float32, precision=lax.Precision.HIGHEST) + br[...]


_proj1 = pl.pallas_call(
    _proj1_body,
    grid=(NP // BN,),
    in_specs=[
        pl.BlockSpec((BN, 48), lambda i: (i, 0)),
        pl.BlockSpec((48, 128), lambda i: (0, 0)),
        pl.BlockSpec((1, 128), lambda i: (0, 0)),
        pl.BlockSpec((128, 512), lambda i: (0, 0)),
        pl.BlockSpec((1, 512), lambda i: (0, 0)),
        pl.BlockSpec((128, 512), lambda i: (0, 0)),
        pl.BlockSpec((1, 512), lambda i: (0, 0)),
    ],
    out_specs=[
        pl.BlockSpec((BN, 512), lambda i: (i, 0)),
        pl.BlockSpec((BN, 512), lambda i: (i, 0)),
    ],
    out_shape=[jax.ShapeDtypeStruct((NP, 512), jnp.float32)] * 2,
)


def _proj2_body(tabs, s1, b1, wl, bl, wr, br, xl, xr):
    t = tabs[...]
    ssum = jnp.sum(s1[...], axis=1)
    hs = [t[h] / (ssum[h][:, None] + 1e-16) for h in range(4)]
    h1 = jnp.maximum(jnp.concatenate(hs, axis=1) + b1[...], 0.0)
    xl[...] = jnp.dot(h1, wl[...], preferred_element_type=jnp.float32, precision=lax.Precision.HIGHEST) + bl[...]
    xr[...] = jnp.dot(h1, wr[...], preferred_element_type=jnp.float32, precision=lax.Precision.HIGHEST) + br[...]


_proj2 = pl.pallas_call(
    _proj2_body,
    grid=(NP // BN,),
    in_specs=[
        pl.BlockSpec((4, BN, 128), lambda i: (0, i, 0)),
        pl.BlockSpec((4, 16, BN), lambda i: (0, 0, i)),
        pl.BlockSpec((1, 512), lambda i: (0, 0)),
        pl.BlockSpec((512, 128), lambda i: (0, 0)),
        pl.BlockSpec((1, 128), lambda i: (0, 0)),
        pl.BlockSpec((512, 128), lambda i: (0, 0)),
        pl.BlockSpec((1, 128), lambda i: (0, 0)),
    ],
    out_specs=[
        pl.BlockSpec((BN, 128), lambda i: (i, 0)),
        pl.BlockSpec((BN, 128), lambda i: (i, 0)),
    ],
    out_shape=[jax.ShapeDtypeStruct((NP, 128), jnp.float32)] * 2,
)


def _fin_body(p, s2, b2, o):
    t = p[...]
    den = jnp.sum(s2[...], axis=0)[:, None] + 1e-16
    o[...] = (t[0] + t[1]) / den + b2[...]


_fin = pl.pallas_call(
    _fin_body,
    grid=(NP // BN,),
    in_specs=[
        pl.BlockSpec((2, BN, 128), lambda i: (0, i, 0)),
        pl.BlockSpec((32, BN), lambda i: (0, i)),
        pl.BlockSpec((1, 128), lambda i: (0, 0)),
    ],
    out_specs=pl.BlockSpec((BN, 128), lambda i: (i, 0)),
    out_shape=jax.ShapeDtypeStruct((NP, 128), jnp.float32),
)


_MESH = plsc.VectorSubcoreMesh(core_axis_name="c", subcore_axis_name="s",
                               num_cores=2, num_subcores=16)
_SC_PARAMS = pltpu.CompilerParams(needs_layout_passes=False)


def _edge_chunk(h_or_none, att_vecs, g, ebuf, gia, gib, gio, qv, lnv,
                Av, Bv, Mv, s_local, ed_hbm, xl_hbm, xr_hbm, acc, semA, semB):
    pltpu.sync_copy(ed_hbm.at[g], ebuf)
    for k in range(4):
        s16 = ebuf[pl.ds(16 * k, 16)]
        d16 = ebuf[pl.ds(B + 16 * k, 16)]
        if h_or_none is None:
            gia[pl.ds(16 * k, 16)] = s16
            gib[pl.ds(16 * k, 16)] = d16
        else:
            gia[pl.ds(16 * k, 16)] = s16 * 4 + h_or_none
            gib[pl.ds(16 * k, 16)] = d16 * 4 + h_or_none
        gio[pl.ds(16 * k, 16)] = d16
        qv[pl.ds(16 * k, 16)] = (d16 >> 4) << 4
        lnv[pl.ds(16 * k, 16)] = d16 & 15
    cpa = pltpu.async_copy(xl_hbm.at[gia], Av, semA)
    cpb = pltpu.async_copy(xr_hbm.at[gib], Bv, semB)
    cpa.wait()
    cpb.wait()
    lanes = lax.iota(jnp.int32, 16)

    @pl.loop(0, B)
    def _score(e):
        accv = None
        for j in range(8):
            t = Av[e, pl.ds(16 * j, 16)] + Bv[e, pl.ds(16 * j, 16)]
            lr = jnp.maximum(t, 0.0) + 0.2 * jnp.minimum(t, 0.0)
            term = lr * att_vecs[j]
            accv = term if accv is None else accv + term
        score = jnp.sum(accv)
        exv = jnp.exp(jnp.zeros((16,), jnp.float32) + score)
        for j in range(8):
            Mv[e, pl.ds(16 * j, 16)] = Av[e, pl.ds(16 * j, 16)] * exv
        q = qv[pl.ds(e, 16)][0]
        ln = lnv[pl.ds(e, 16)][0]
        oh = (lanes == ln).astype(jnp.float32)
        s_local[pl.ds(q, 16)] = s_local[pl.ds(q, 16)] + oh * exv

    pltpu.sync_copy(Mv, acc.at[gio], add=True)


def _zero_slocal(s_local):
    z16 = jnp.zeros((16,), jnp.float32)

    @pl.loop(0, NP // 16)
    def _(i):
        s_local[pl.ds(i * 16, 16)] = z16


def _zero_rows(ref):
    z16 = jnp.zeros((16,), jnp.float32)

    @pl.loop(0, ref.shape[0])
    def _(r):
        for j in range(ref.shape[1] // 16):
            ref[r, pl.ds(16 * j, 16)] = z16


_EPT1 = E_PAD // 16
_NCH1 = _EPT1 // B


@functools.partial(
    pl.kernel,
    out_type=[jax.ShapeDtypeStruct((4, NP, 128), jnp.float32),
              jax.ShapeDtypeStruct((4, 16, NP), jnp.float32)],
    mesh=_MESH,
    compiler_params=_SC_PARAMS,
    scratch_types=[
        pltpu.VMEM((2 * B,), jnp.int32),
        pltpu.VMEM((4, 128), jnp.float32),
        pltpu.VMEM((B,), jnp.int32),
        pltpu.VMEM((B,), jnp.int32),
        pltpu.VMEM((B,), jnp.int32),
        pltpu.VMEM((B + 16,), jnp.int32),
        pltpu.VMEM((B + 16,), jnp.int32),
        pltpu.VMEM((B, 128), jnp.float32),
        pltpu.VMEM((B, 128), jnp.float32),
        pltpu.VMEM((B, 128), jnp.float32),
        pltpu.VMEM((NP,), jnp.float32),
        pltpu.VMEM_SHARED((NP, 128), jnp.float32),
        pltpu.SemaphoreType.DMA,
        pltpu.SemaphoreType.DMA,
    ],
)
def _sc_gat1(xl_hbm, xr_hbm, ed_hbm, att_hbm,
             out_hbm, sout_hbm,
             ebuf, attv, gia, gib, gio, qv, lnv, Av, Bv, Mv, s_local,
             acc, semA, semB):
    cid = lax.axis_index("c")
    sid = lax.axis_index("s")
    pltpu.sync_copy(att_hbm, attv)
    rbase = sid * (NP // 16)
    for hh in range(2):
        h = cid * 2 + hh
        att_vecs = [attv[h, pl.ds(16 * j, 16)] for j in range(8)]
        _zero_slocal(s_local)
        _zero_rows(Mv)
        for k in range(STRC):
            pltpu.sync_copy(Mv, acc.at[pl.ds(rbase + B * k, B)])
        plsc.subcore_barrier()

        @pl.loop(0, _NCH1)
        def _chunk(c):
            _edge_chunk(h, att_vecs, sid * _NCH1 + c, ebuf, gia, gib, gio,
                        qv, lnv, Av, Bv, Mv, s_local, ed_hbm, xl_hbm, xr_hbm,
                        acc, semA, semB)

        plsc.subcore_barrier()
        for k in range(STRC):
            sl = pl.ds(rbase + B * k, B)
            pltpu.sync_copy(acc.at[sl], out_hbm.at[h, sl])
        pltpu.sync_copy(s_local, sout_hbm.at[h, sid])
        plsc.subcore_barrier()


_EPT2 = E_PAD // 32
_NCH2 = _EPT2 // B


@functools.partial(
    pl.kernel,
    out_type=[jax.ShapeDtypeStruct((2, NP, 128), jnp.float32),
              jax.ShapeDtypeStruct((32, NP), jnp.float32)],
    mesh=_MESH,
    compiler_params=_SC_PARAMS,
    scratch_types=[
        pltpu.VMEM((2 * B,), jnp.int32),
        pltpu.VMEM((1, 128), jnp.float32),
        pltpu.VMEM((B,), jnp.int32),
        pltpu.VMEM((B,), jnp.int32),
        pltpu.VMEM((B,), jnp.int32),
        pltpu.VMEM((B + 16,), jnp.int32),
        pltpu.VMEM((B + 16,), jnp.int32),
        pltpu.VMEM((B, 128), jnp.float32),
        pltpu.VMEM((B, 128), jnp.float32),
        pltpu.VMEM((B, 128), jnp.float32),
        pltpu.VMEM((NP,), jnp.float32),
        pltpu.VMEM_SHARED((NP, 128), jnp.float32),
        pltpu.SemaphoreType.DMA,
        pltpu.SemaphoreType.DMA,
    ],
)
def _sc_gat2(xl_hbm, xr_hbm, ed_hbm, att_hbm,
             out_hbm, sout_hbm,
             ebuf, attv, gia, gib, gio, qv, lnv, Av, Bv, Mv, s_local,
             acc, semA, semB):
    cid = lax.axis_index("c")
    sid = lax.axis_index("s")
    wid = sid * 2 + cid
    pltpu.sync_copy(att_hbm, attv)
    _zero_rows(Mv)
    rbase = sid * (NP // 16)
    att_vecs = [attv[0, pl.ds(16 * j, 16)] for j in range(8)]
    _zero_slocal(s_local)
    for k in range(STRC):
        pltpu.sync_copy(Mv, acc.at[pl.ds(rbase + B * k, B)])
    plsc.subcore_barrier()

    @pl.loop(0, _NCH2)
    def _chunk(c):
        _edge_chunk(None, att_vecs, wid * _NCH2 + c, ebuf, gia, gib, gio,
                    qv, lnv, Av, Bv, Mv, s_local, ed_hbm, xl_hbm, xr_hbm,
                    acc, semA, semB)

    plsc.subcore_barrier()
    for k in range(STRC):
        sl = pl.ds(rbase + B * k, B)
        pltpu.sync_copy(acc.at[sl], out_hbm.at[cid, sl])
    pltpu.sync_copy(s_local, sout_hbm.at[wid])
    plsc.subcore_barrier()


def kernel(x, edge_index, conv_w, conv_b, W_l1, b_l1, W_r1, b_r1, att1, bias1,
           W_l2, b_l2, W_r2, b_r2, att2, bias2):
    xflat = jnp.transpose(x, (0, 2, 1)).reshape(N, 48)
    xpad = jnp.zeros((NP, 48), jnp.float32).at[:N].set(xflat)
    wc = conv_w[:, 0, :]
    wbig = jnp.zeros((48, 128), jnp.float32)
    wbig = wbig.at[0:24, 0::2].set(wc.T)
    wbig = wbig.at[24:48, 1::2].set(wc.T)
    cbvec = jnp.repeat(conv_b, 2).reshape(1, 128)

    xl1, xr1 = _proj1(xpad, wbig, cbvec, W_l1, b_l1.reshape(1, 512),
                      W_r1, b_r1.reshape(1, 512))

    loop = jnp.arange(N, dtype=jnp.int32)
    padi = jnp.full((E_PAD - E_RAW - N,), N, jnp.int32)
    src = jnp.concatenate([edge_index[0].astype(jnp.int32), loop, padi])
    dst = jnp.concatenate([edge_index[1].astype(jnp.int32), loop, padi])
    ed = jnp.concatenate([src.reshape(-1, B), dst.reshape(-1, B)], axis=1)

    tabs, s1 = _sc_gat1(xl1.reshape(NP * 4, 128), xr1.reshape(NP * 4, 128),
                        ed, att1)

    xl2, xr2 = _proj2(tabs, s1, bias1.reshape(1, 512), W_l2,
                      b_l2.reshape(1, 128), W_r2, b_r2.reshape(1, 128))

    parts, s2 = _sc_gat2(xl2, xr2, ed, att2)

    out = _fin(parts, s2, bias2.reshape(1, 128))
    return out[:N]


# speedup vs baseline: 11.7127x; 1.3410x over previous
import functools

import jax
import jax.numpy as jnp
from jax import lax
from jax.experimental import pallas as pl
from jax.experimental.pallas import tpu as pltpu
from jax.experimental.pallas import tpu_sc as plsc

N = 10000
NP = 10240
E_RAW = 320000
E_PAD = 331776
B = 48
BN = 1024


def _proj1_body(x48, wbig, cb, wl, bl, wr, br, xl, xr):
    xf = jnp.dot(x48[...], wbig[...], preferred_element_type=jnp.float32, precision=lax.Precision.HIGHEST)
    xf = xf + cb[...]
    xl[...] = jnp.dot(xf, wl[...], preferred_element_type=jnp.float32, precision=lax.Precision.HIGHEST) + bl[...]
    xr[...] = jnp.dot(xf, wr[...], preferred_element_type=jnp.float32, precision=lax.Precision.HIGHEST) + br[...]


_proj1 = pl.pallas_call(
    _proj1_body,
    grid=(NP // BN,),
    in_specs=[
        pl.BlockSpec((BN, 48), lambda i: (i, 0)),
        pl.BlockSpec((48, 128), lambda i: (0, 0)),
        pl.BlockSpec((1, 128), lambda i: (0, 0)),
        pl.BlockSpec((128, 512), lambda i: (0, 0)),
        pl.BlockSpec((1, 512), lambda i: (0, 0)),
        pl.BlockSpec((128, 512), lambda i: (0, 0)),
        pl.BlockSpec((1, 512), lambda i: (0, 0)),
    ],
    out_specs=[
        pl.BlockSpec((BN, 512), lambda i: (i, 0)),
        pl.BlockSpec((BN, 512), lambda i: (i, 0)),
    ],
    out_shape=[jax.ShapeDtypeStruct((NP, 512), jnp.float32)] * 2,
)


def _proj2_body(tabs, s1, b1, wl, bl, wr, br, xl, xr):
    t = tabs[...]
    ssum = jnp.sum(s1[...], axis=1)
    hs = [t[h] / (ssum[h][:, None] + 1e-16) for h in range(4)]
    h1 = jnp.maximum(jnp.concatenate(hs, axis=1) + b1[...], 0.0)
    xl[...] = jnp.dot(h1, wl[...], preferred_element_type=jnp.float32, precision=lax.Precision.HIGHEST) + bl[...]
    xr[...] = jnp.dot(h1, wr[...], preferred_element_type=jnp.float32, precision=lax.Precision.HIGHEST) + br[...]


_proj2 = pl.pallas_call(
    _proj2_body,
    grid=(NP // BN,),
    in_specs=[
        pl.BlockSpec((4, BN, 128), lambda i: (0, i, 0)),
        pl.BlockSpec((4, 16, BN), lambda i: (0, 0, i)),
        pl.BlockSpec((1, 512), lambda i: (0, 0)),
        pl.BlockSpec((512, 128), lambda i: (0, 0)),
        pl.BlockSpec((1, 128), lambda i: (0, 0)),
        pl.BlockSpec((512, 128), lambda i: (0, 0)),
        pl.BlockSpec((1, 128), lambda i: (0, 0)),
    ],
    out_specs=[
        pl.BlockSpec((BN, 128), lambda i: (i, 0)),
        pl.BlockSpec((BN, 128), lambda i: (i, 0)),
    ],
    out_shape=[jax.ShapeDtypeStruct((NP, 128), jnp.float32)] * 2,
)


def _fin_body(p, s2, b2, o):
    t = p[...]
    den = jnp.sum(s2[...], axis=0)[:, None] + 1e-16
    o[...] = (t[0] + t[1]) / den + b2[...]


_fin = pl.pallas_call(
    _fin_body,
    grid=(NP // BN,),
    in_specs=[
        pl.BlockSpec((2, BN, 128), lambda i: (0, i, 0)),
        pl.BlockSpec((32, BN), lambda i: (0, i)),
        pl.BlockSpec((1, 128), lambda i: (0, 0)),
    ],
    out_specs=pl.BlockSpec((BN, 128), lambda i: (i, 0)),
    out_shape=jax.ShapeDtypeStruct((NP, 128), jnp.float32),
)


_MESH = plsc.VectorSubcoreMesh(core_axis_name="c", subcore_axis_name="s",
                               num_cores=2, num_subcores=16)
_SC_PARAMS = pltpu.CompilerParams(needs_layout_passes=False)


def _idx_phase(h_or_none, g, ebuf, row, gia, gib, gio, qv, lnv):
    for k in range(B // 16):
        s16 = ebuf[row, pl.ds(16 * k, 16)]
        d16 = ebuf[row, pl.ds(B + 16 * k, 16)]
        if h_or_none is None:
            gia[pl.ds(16 * k, 16)] = s16
            gib[pl.ds(16 * k, 16)] = d16
        else:
            gia[pl.ds(16 * k, 16)] = s16 * 4 + h_or_none
            gib[pl.ds(16 * k, 16)] = d16 * 4 + h_or_none
        gio[pl.ds(16 * k, 16)] = d16
        qv[pl.ds(16 * k, 16)] = (d16 >> 4) << 4
        lnv[pl.ds(16 * k, 16)] = d16 & 15


def _compute_chunk(att_vecs, gio, qv, lnv, Av, Bv, Mv, s_local, acc):
    lanes = lax.iota(jnp.int32, 16)

    @pl.loop(0, B)
    def _score(e):
        accv = None
        for j in range(8):
            t = Av[e, pl.ds(16 * j, 16)] + Bv[e, pl.ds(16 * j, 16)]
            lr = jnp.maximum(t, 0.0) + 0.2 * jnp.minimum(t, 0.0)
            term = lr * att_vecs[j]
            accv = term if accv is None else accv + term
        score = jnp.sum(accv)
        exv = jnp.exp(jnp.zeros((16,), jnp.float32) + score)
        for j in range(8):
            Mv[e, pl.ds(16 * j, 16)] = Av[e, pl.ds(16 * j, 16)] * exv
        q = qv[pl.ds(e, 16)][0]
        ln = lnv[pl.ds(e, 16)][0]
        oh = (lanes == ln).astype(jnp.float32)
        s_local[pl.ds(q, 16)] = s_local[pl.ds(q, 16)] + oh * exv

    pltpu.sync_copy(Mv, acc.at[gio], add=True)


def _edge_pass(h_or_none, nch, tbase, att_vecs, ed_hbm, xl_hbm, xr_hbm,
               ebuf, gia, gib, gio, qv, lnv, Av, Bv, Mv, s_local, acc,
               semE, semA, semB):
    def issue(sl, cp=pltpu.async_copy):
        cp(xl_hbm.at[gia.at[sl]], Av.at[sl], semA.at[sl])
        cp(xr_hbm.at[gib.at[sl]], Bv.at[sl], semB.at[sl])

    def wait(sl):
        pltpu.make_async_copy(xl_hbm.at[gia.at[sl]], Av.at[sl],
                              semA.at[sl]).wait()
        pltpu.make_async_copy(xr_hbm.at[gib.at[sl]], Bv.at[sl],
                              semB.at[sl]).wait()

    pltpu.sync_copy(ed_hbm.at[pl.ds(tbase, 2)], ebuf.at[0])
    for sl in range(2):
        _idx_phase(h_or_none, None, ebuf.at[0], sl, gia.at[sl], gib.at[sl],
                   gio.at[sl], qv.at[sl], lnv.at[sl])
        issue(sl)

    @pl.loop(0, nch, step=2)
    def _body(g):
        p = (g // 2) & 1
        pltpu.async_copy(ed_hbm.at[pl.ds(tbase + g + 2, 2)], ebuf.at[1 - p],
                         semE)
        wait(0)
        _compute_chunk(att_vecs, gio.at[0], qv.at[0], lnv.at[0], Av.at[0],
                       Bv.at[0], Mv, s_local, acc)
        pltpu.make_async_copy(ed_hbm.at[pl.ds(tbase + g + 2, 2)],
                              ebuf.at[1 - p], semE).wait()
        _idx_phase(h_or_none, None, ebuf.at[1 - p], 0, gia.at[0], gib.at[0],
                   gio.at[0], qv.at[0], lnv.at[0])
        issue(0)
        wait(1)
        _compute_chunk(att_vecs, gio.at[1], qv.at[1], lnv.at[1], Av.at[1],
                       Bv.at[1], Mv, s_local, acc)
        _idx_phase(h_or_none, None, ebuf.at[1 - p], 1, gia.at[1], gib.at[1],
                   gio.at[1], qv.at[1], lnv.at[1])
        issue(1)

    wait(0)
    wait(1)


def _zero_slocal(s_local):
    z16 = jnp.zeros((16,), jnp.float32)

    @pl.loop(0, NP // 16)
    def _(i):
        s_local[pl.ds(i * 16, 16)] = z16


def _zero_rows(ref):
    z16 = jnp.zeros((16,), jnp.float32)

    @pl.loop(0, ref.shape[0])
    def _(r):
        for j in range(ref.shape[1] // 16):
            ref[r, pl.ds(16 * j, 16)] = z16


def _zero_stripe(Mv, dst, rbase):
    nfull = (NP // 16) // B * B
    for k in range((NP // 16) // B):
        pltpu.sync_copy(Mv, dst.at[pl.ds(rbase + B * k, B)])
    rem = NP // 16 - nfull
    if rem:
        pltpu.sync_copy(Mv.at[pl.ds(0, rem)],
                        dst.at[pl.ds(rbase + nfull, rem)])


def _write_stripe(acc, out, rbase):
    nfull = (NP // 16) // B * B
    for k in range((NP // 16) // B):
        sl = pl.ds(rbase + B * k, B)
        pltpu.sync_copy(acc.at[sl], out.at[sl])
    rem = NP // 16 - nfull
    if rem:
        sl = pl.ds(rbase + nfull, rem)
        pltpu.sync_copy(acc.at[sl], out.at[sl])


_NCH1 = E_PAD // 16 // B


@functools.partial(
    pl.kernel,
    out_type=[jax.ShapeDtypeStruct((4, NP, 128), jnp.float32),
              jax.ShapeDtypeStruct((4, 16, NP), jnp.float32)],
    mesh=_MESH,
    compiler_params=_SC_PARAMS,
    scratch_types=[
        pltpu.VMEM((2, 2, 2 * B), jnp.int32),
        pltpu.VMEM((4, 128), jnp.float32),
        pltpu.VMEM((2, B), jnp.int32),
        pltpu.VMEM((2, B), jnp.int32),
        pltpu.VMEM((2, B), jnp.int32),
        pltpu.VMEM((2, B + 16), jnp.int32),
        pltpu.VMEM((2, B + 16), jnp.int32),
        pltpu.VMEM((2, B, 128), jnp.float32),
        pltpu.VMEM((2, B, 128), jnp.float32),
        pltpu.VMEM((B, 128), jnp.float32),
        pltpu.VMEM((NP,), jnp.float32),
        pltpu.VMEM_SHARED((NP, 128), jnp.float32),
        pltpu.SemaphoreType.DMA,
        pltpu.SemaphoreType.DMA((2,)),
        pltpu.SemaphoreType.DMA((2,)),
    ],
)
def _sc_gat1(xl_hbm, xr_hbm, ed_hbm, att_hbm, out_hbm, sout_hbm,
             ebuf, attv, gia, gib, gio, qv, lnv, Av, Bv, Mv, s_local,
             acc, semE, semA, semB):
    cid = lax.axis_index("c")
    sid = lax.axis_index("s")
    pltpu.sync_copy(att_hbm, attv)
    rbase = sid * (NP // 16)
    for hh in range(2):
        h = cid * 2 + hh
        att_vecs = [attv[h, pl.ds(16 * j, 16)] for j in range(8)]
        _zero_slocal(s_local)
        _zero_rows(Mv)
        _zero_stripe(Mv, acc, rbase)
        plsc.subcore_barrier()
        _edge_pass(h, _NCH1, sid * _NCH1, att_vecs, ed_hbm, xl_hbm, xr_hbm,
                   ebuf, gia, gib, gio, qv, lnv, Av, Bv, Mv, s_local, acc,
                   semE, semA, semB)
        plsc.subcore_barrier()
        _write_stripe(acc, out_hbm.at[h], rbase)
        pltpu.sync_copy(s_local, sout_hbm.at[h, sid])
        plsc.subcore_barrier()


_NCH2 = E_PAD // 32 // B


@functools.partial(
    pl.kernel,
    out_type=[jax.ShapeDtypeStruct((2, NP, 128), jnp.float32),
              jax.ShapeDtypeStruct((32, NP), jnp.float32)],
    mesh=_MESH,
    compiler_params=_SC_PARAMS,
    scratch_types=[
        pltpu.VMEM((2, 2, 2 * B), jnp.int32),
        pltpu.VMEM((1, 128), jnp.float32),
        pltpu.VMEM((2, B), jnp.int32),
        pltpu.VMEM((2, B), jnp.int32),
        pltpu.VMEM((2, B), jnp.int32),
        pltpu.VMEM((2, B + 16), jnp.int32),
        pltpu.VMEM((2, B + 16), jnp.int32),
        pltpu.VMEM((2, B, 128), jnp.float32),
        pltpu.VMEM((2, B, 128), jnp.float32),
        pltpu.VMEM((B, 128), jnp.float32),
        pltpu.VMEM((NP,), jnp.float32),
        pltpu.VMEM_SHARED((NP, 128), jnp.float32),
        pltpu.SemaphoreType.DMA,
        pltpu.SemaphoreType.DMA((2,)),
        pltpu.SemaphoreType.DMA((2,)),
    ],
)
def _sc_gat2(xl_hbm, xr_hbm, ed_hbm, att_hbm, out_hbm, sout_hbm,
             ebuf, attv, gia, gib, gio, qv, lnv, Av, Bv, Mv, s_local,
             acc, semE, semA, semB):
    cid = lax.axis_index("c")
    sid = lax.axis_index("s")
    wid = sid * 2 + cid
    pltpu.sync_copy(att_hbm, attv)
    rbase = sid * (NP // 16)
    att_vecs = [attv[0, pl.ds(16 * j, 16)] for j in range(8)]
    _zero_slocal(s_local)
    _zero_rows(Mv)
    _zero_stripe(Mv, acc, rbase)
    plsc.subcore_barrier()
    _edge_pass(None, _NCH2, wid * _NCH2, att_vecs, ed_hbm, xl_hbm, xr_hbm,
               ebuf, gia, gib, gio, qv, lnv, Av, Bv, Mv, s_local, acc,
               semE, semA, semB)
    plsc.subcore_barrier()
    _write_stripe(acc, out_hbm.at[cid], rbase)
    pltpu.sync_copy(s_local, sout_hbm.at[wid])
    plsc.subcore_barrier()


def kernel(x, edge_index, conv_w, conv_b, W_l1, b_l1, W_r1, b_r1, att1, bias1,
           W_l2, b_l2, W_r2, b_r2, att2, bias2):
    xflat = jnp.transpose(x, (0, 2, 1)).reshape(N, 48)
    xpad = jnp.zeros((NP, 48), jnp.float32).at[:N].set(xflat)
    wc = conv_w[:, 0, :]
    wbig = jnp.zeros((48, 128), jnp.float32)
    wbig = wbig.at[0:24, 0::2].set(wc.T)
    wbig = wbig.at[24:48, 1::2].set(wc.T)
    cbvec = jnp.repeat(conv_b, 2).reshape(1, 128)

    xl1, xr1 = _proj1(xpad, wbig, cbvec, W_l1, b_l1.reshape(1, 512),
                      W_r1, b_r1.reshape(1, 512))

    loop = jnp.arange(N, dtype=jnp.int32)
    padi = jnp.full((E_PAD - E_RAW - N,), N, jnp.int32)
    src = jnp.concatenate([edge_index[0].astype(jnp.int32), loop, padi])
    dst = jnp.concatenate([edge_index[1].astype(jnp.int32), loop, padi])
    ed = jnp.concatenate([src.reshape(-1, B), dst.reshape(-1, B)], axis=1)
    ed = jnp.concatenate([ed, jnp.full((2, 2 * B), N, jnp.int32)], axis=0)

    tabs, s1 = _sc_gat1(xl1.reshape(NP * 4, 128), xr1.reshape(NP * 4, 128),
                        ed, att1)

    xl2, xr2 = _proj2(tabs, s1, bias1.reshape(1, 512), W_l2,
                      b_l2.reshape(1, 128), W_r2, b_r2.reshape(1, 128))

    parts, s2 = _sc_gat2(xl2, xr2, ed, att2)

    out = _fin(parts, s2, bias2.reshape(1, 128))
    return out[:N]


# speedup vs baseline: 18.1909x; 1.5531x over previous
import functools

import jax
import jax.numpy as jnp
from jax import lax
from jax.experimental import pallas as pl
from jax.experimental.pallas import tpu as pltpu
from jax.experimental.pallas import tpu_sc as plsc

N = 10000
NP = 10240
E_RAW = 320000
E_PAD = 331776
B = 48
BN = 1024


def _proj1_body(x48, wbig, cb, wl, bl, wr, br, xl, xr):
    xf = jnp.dot(x48[...], wbig[...], preferred_element_type=jnp.float32, precision=lax.Precision.HIGHEST)
    xf = xf + cb[...]
    xl[...] = jnp.dot(xf, wl[...], preferred_element_type=jnp.float32, precision=lax.Precision.HIGHEST) + bl[...]
    xr[...] = jnp.dot(xf, wr[...], preferred_element_type=jnp.float32, precision=lax.Precision.HIGHEST) + br[...]


_proj1 = pl.pallas_call(
    _proj1_body,
    grid=(NP // BN,),
    in_specs=[
        pl.BlockSpec((BN, 48), lambda i: (i, 0)),
        pl.BlockSpec((48, 128), lambda i: (0, 0)),
        pl.BlockSpec((1, 128), lambda i: (0, 0)),
        pl.BlockSpec((128, 512), lambda i: (0, 0)),
        pl.BlockSpec((1, 512), lambda i: (0, 0)),
        pl.BlockSpec((128, 512), lambda i: (0, 0)),
        pl.BlockSpec((1, 512), lambda i: (0, 0)),
    ],
    out_specs=[
        pl.BlockSpec((BN, 512), lambda i: (i, 0)),
        pl.BlockSpec((BN, 512), lambda i: (i, 0)),
    ],
    out_shape=[jax.ShapeDtypeStruct((NP, 512), jnp.float32)] * 2,
)


def _proj2_body(tabs, s1, b1, wl, bl, wr, br, xl, xr):
    t = tabs[...]
    ssum = jnp.sum(s1[...], axis=1)
    hs = [t[h] / (ssum[h][:, None] + 1e-16) for h in range(4)]
    h1 = jnp.maximum(jnp.concatenate(hs, axis=1) + b1[...], 0.0)
    xl[...] = jnp.dot(h1, wl[...], preferred_element_type=jnp.float32, precision=lax.Precision.HIGHEST) + bl[...]
    xr[...] = jnp.dot(h1, wr[...], preferred_element_type=jnp.float32, precision=lax.Precision.HIGHEST) + br[...]


_proj2 = pl.pallas_call(
    _proj2_body,
    grid=(NP // BN,),
    in_specs=[
        pl.BlockSpec((4, BN, 128), lambda i: (0, i, 0)),
        pl.BlockSpec((4, 16, BN), lambda i: (0, 0, i)),
        pl.BlockSpec((1, 512), lambda i: (0, 0)),
        pl.BlockSpec((512, 128), lambda i: (0, 0)),
        pl.BlockSpec((1, 128), lambda i: (0, 0)),
        pl.BlockSpec((512, 128), lambda i: (0, 0)),
        pl.BlockSpec((1, 128), lambda i: (0, 0)),
    ],
    out_specs=[
        pl.BlockSpec((BN, 128), lambda i: (i, 0)),
        pl.BlockSpec((BN, 128), lambda i: (i, 0)),
    ],
    out_shape=[jax.ShapeDtypeStruct((NP, 128), jnp.float32)] * 2,
)


def _fin_body(p, s2, b2, o):
    t = p[...]
    den = jnp.sum(s2[...], axis=0)[:, None] + 1e-16
    o[...] = (t[0] + t[1]) / den + b2[...]


_fin = pl.pallas_call(
    _fin_body,
    grid=(NP // BN,),
    in_specs=[
        pl.BlockSpec((2, BN, 128), lambda i: (0, i, 0)),
        pl.BlockSpec((32, BN), lambda i: (0, i)),
        pl.BlockSpec((1, 128), lambda i: (0, 0)),
    ],
    out_specs=pl.BlockSpec((BN, 128), lambda i: (i, 0)),
    out_shape=jax.ShapeDtypeStruct((NP, 128), jnp.float32),
)


_MESH = plsc.VectorSubcoreMesh(core_axis_name="c", subcore_axis_name="s",
                               num_cores=2, num_subcores=16)
_SC_PARAMS = pltpu.CompilerParams(needs_layout_passes=False)


def _idx_phase(h_or_none, ebuf, row, gia, gib, gio, dstw):
    for k in range(B // 16):
        s16 = ebuf[row, pl.ds(16 * k, 16)]
        d16 = ebuf[row, pl.ds(B + 16 * k, 16)]
        if h_or_none is None:
            gia[pl.ds(16 * k, 16)] = s16
            gib[pl.ds(16 * k, 16)] = d16
        else:
            gia[pl.ds(16 * k, 16)] = s16 * 4 + h_or_none
            gib[pl.ds(16 * k, 16)] = d16 * 4 + h_or_none
        gio[pl.ds(16 * k, 16)] = d16
        dstw[pl.ds(16 * k, 16)] = d16


def _compute_chunk(att06, att04, gio, dstw, Av, Bv, Mv, exb, s_local, acc,
                   semS):
    lanes = lax.iota(jnp.int32, 16)

    @plsc.parallel_loop(0, B, unroll=1)
    def _score(e):
        accv = None
        for j in range(8):
            a = Av[e, pl.ds(16 * j, 16)]
            t = a + Bv[e, pl.ds(16 * j, 16)]
            u1 = t * att06[j]
            u2 = jnp.abs(t) * att04[j]
            accv = u1 + u2 if accv is None else accv + u1 + u2
        score = jnp.sum(accv)
        exv = jnp.exp(jnp.zeros((16,), jnp.float32) + score)
        for j in range(8):
            Mv[e, pl.ds(16 * j, 16)] = Av[e, pl.ds(16 * j, 16)] * exv
        exb[e, pl.ds(0, 16)] = exv

    cps = pltpu.async_copy(Mv, acc.at[gio], semS, add=True)

    @pl.loop(0, B)
    def _spass(e):
        d = dstw[pl.ds(e, 16)][0]
        q = (d >> 4) << 4
        ln = d & 15
        oh = (lanes == ln).astype(jnp.float32)
        exv = exb[e, pl.ds(0, 16)]
        s_local[pl.ds(q, 16)] = s_local[pl.ds(q, 16)] + oh * exv

    cps.wait()


def _edge_pass(h_or_none, nch, tbase, att06, att04, ed_hbm, xl_hbm, xr_hbm,
               ebuf, gia, gib, gio, dstw, Av, Bv, Mv, exb, s_local, acc,
               semE, semA, semB, semS):
    def issue(sl):
        pltpu.async_copy(xl_hbm.at[gia.at[sl]], Av.at[sl], semA.at[sl])
        pltpu.async_copy(xr_hbm.at[gib.at[sl]], Bv.at[sl], semB.at[sl])

    def wait(sl):
        pltpu.make_async_copy(xl_hbm.at[gia.at[sl]], Av.at[sl],
                              semA.at[sl]).wait()
        pltpu.make_async_copy(xr_hbm.at[gib.at[sl]], Bv.at[sl],
                              semB.at[sl]).wait()

    def compute(sl):
        _compute_chunk(att06, att04, gio.at[sl], dstw.at[sl], Av.at[sl],
                       Bv.at[sl], Mv, exb, s_local, acc, semS)

    pltpu.sync_copy(ed_hbm.at[pl.ds(tbase, 2)], ebuf.at[0])
    for sl in range(2):
        _idx_phase(h_or_none, ebuf.at[0], sl, gia.at[sl], gib.at[sl],
                   gio.at[sl], dstw.at[sl])
        issue(sl)

    @pl.loop(0, nch, step=2)
    def _body(g):
        p = (g // 2) & 1
        pltpu.async_copy(ed_hbm.at[pl.ds(tbase + g + 2, 2)], ebuf.at[1 - p],
                         semE)
        wait(0)
        compute(0)
        pltpu.make_async_copy(ed_hbm.at[pl.ds(tbase + g + 2, 2)],
                              ebuf.at[1 - p], semE).wait()
        _idx_phase(h_or_none, ebuf.at[1 - p], 0, gia.at[0], gib.at[0],
                   gio.at[0], dstw.at[0])
        issue(0)
        wait(1)
        compute(1)
        _idx_phase(h_or_none, ebuf.at[1 - p], 1, gia.at[1], gib.at[1],
                   gio.at[1], dstw.at[1])
        issue(1)

    wait(0)
    wait(1)


def _zero_slocal(s_local):
    z16 = jnp.zeros((16,), jnp.float32)

    @pl.loop(0, NP // 16)
    def _(i):
        s_local[pl.ds(i * 16, 16)] = z16


def _zero_rows(ref):
    z16 = jnp.zeros((16,), jnp.float32)

    @pl.loop(0, ref.shape[0])
    def _(r):
        for j in range(ref.shape[1] // 16):
            ref[r, pl.ds(16 * j, 16)] = z16


def _zero_stripe(Mv, dst, rbase):
    nfull = (NP // 16) // B * B
    for k in range((NP // 16) // B):
        pltpu.sync_copy(Mv, dst.at[pl.ds(rbase + B * k, B)])
    rem = NP // 16 - nfull
    if rem:
        pltpu.sync_copy(Mv.at[pl.ds(0, rem)],
                        dst.at[pl.ds(rbase + nfull, rem)])


def _write_stripe(acc, out, rbase):
    nfull = (NP // 16) // B * B
    for k in range((NP // 16) // B):
        sl = pl.ds(rbase + B * k, B)
        pltpu.sync_copy(acc.at[sl], out.at[sl])
    rem = NP // 16 - nfull
    if rem:
        sl = pl.ds(rbase + nfull, rem)
        pltpu.sync_copy(acc.at[sl], out.at[sl])


_NCH1 = E_PAD // 16 // B


@functools.partial(
    pl.kernel,
    out_type=[jax.ShapeDtypeStruct((4, NP, 128), jnp.float32),
              jax.ShapeDtypeStruct((4, 16, NP), jnp.float32)],
    mesh=_MESH,
    compiler_params=_SC_PARAMS,
    scratch_types=[
        pltpu.VMEM((2, 2, 2 * B), jnp.int32),
        pltpu.VMEM((4, 128), jnp.float32),
        pltpu.VMEM((2, B), jnp.int32),
        pltpu.VMEM((2, B), jnp.int32),
        pltpu.VMEM((2, B), jnp.int32),
        pltpu.VMEM((2, B + 16), jnp.int32),
        pltpu.VMEM((2, B, 128), jnp.float32),
        pltpu.VMEM((2, B, 128), jnp.float32),
        pltpu.VMEM((B, 128), jnp.float32),
        pltpu.VMEM((B, 16), jnp.float32),
        pltpu.VMEM((NP,), jnp.float32),
        pltpu.VMEM_SHARED((NP, 128), jnp.float32),
        pltpu.SemaphoreType.DMA,
        pltpu.SemaphoreType.DMA((2,)),
        pltpu.SemaphoreType.DMA((2,)),
        pltpu.SemaphoreType.DMA,
    ],
)
def _sc_gat1(xl_hbm, xr_hbm, ed_hbm, att_hbm, out_hbm, sout_hbm,
             ebuf, attv, gia, gib, gio, dstw, Av, Bv, Mv, exb, s_local,
             acc, semE, semA, semB, semS):
    cid = lax.axis_index("c")
    sid = lax.axis_index("s")
    pltpu.sync_copy(att_hbm, attv)
    rbase = sid * (NP // 16)
    for hh in range(2):
        h = cid * 2 + hh
        att06 = [attv[h, pl.ds(16 * j, 16)] * 0.6 for j in range(8)]
        att04 = [attv[h, pl.ds(16 * j, 16)] * 0.4 for j in range(8)]
        _zero_slocal(s_local)
        _zero_rows(Mv)
        _zero_stripe(Mv, acc, rbase)
        plsc.subcore_barrier()
        _edge_pass(h, _NCH1, sid * _NCH1, att06, att04, ed_hbm, xl_hbm,
                   xr_hbm, ebuf, gia, gib, gio, dstw, Av, Bv, Mv, exb,
                   s_local, acc, semE, semA, semB, semS)
        plsc.subcore_barrier()
        _write_stripe(acc, out_hbm.at[h], rbase)
        pltpu.sync_copy(s_local, sout_hbm.at[h, sid])
        plsc.subcore_barrier()


_NCH2 = E_PAD // 32 // B


@functools.partial(
    pl.kernel,
    out_type=[jax.ShapeDtypeStruct((2, NP, 128), jnp.float32),
              jax.ShapeDtypeStruct((32, NP), jnp.float32)],
    mesh=_MESH,
    compiler_params=_SC_PARAMS,
    scratch_types=[
        pltpu.VMEM((2, 2, 2 * B), jnp.int32),
        pltpu.VMEM((1, 128), jnp.float32),
        pltpu.VMEM((2, B), jnp.int32),
        pltpu.VMEM((2, B), jnp.int32),
        pltpu.VMEM((2, B), jnp.int32),
        pltpu.VMEM((2, B + 16), jnp.int32),
        pltpu.VMEM((2, B, 128), jnp.float32),
        pltpu.VMEM((2, B, 128), jnp.float32),
        pltpu.VMEM((B, 128), jnp.float32),
        pltpu.VMEM((B, 16), jnp.float32),
        pltpu.VMEM((NP,), jnp.float32),
        pltpu.VMEM_SHARED((NP, 128), jnp.float32),
        pltpu.SemaphoreType.DMA,
        pltpu.SemaphoreType.DMA((2,)),
        pltpu.SemaphoreType.DMA((2,)),
        pltpu.SemaphoreType.DMA,
    ],
)
def _sc_gat2(xl_hbm, xr_hbm, ed_hbm, att_hbm, out_hbm, sout_hbm,
             ebuf, attv, gia, gib, gio, dstw, Av, Bv, Mv, exb, s_local,
             acc, semE, semA, semB, semS):
    cid = lax.axis_index("c")
    sid = lax.axis_index("s")
    wid = sid * 2 + cid
    pltpu.sync_copy(att_hbm, attv)
    rbase = sid * (NP // 16)
    att06 = [attv[0, pl.ds(16 * j, 16)] * 0.6 for j in range(8)]
    att04 = [attv[0, pl.ds(16 * j, 16)] * 0.4 for j in range(8)]
    _zero_slocal(s_local)
    _zero_rows(Mv)
    _zero_stripe(Mv, acc, rbase)
    plsc.subcore_barrier()
    _edge_pass(None, _NCH2, wid * _NCH2, att06, att04, ed_hbm, xl_hbm,
               xr_hbm, ebuf, gia, gib, gio, dstw, Av, Bv, Mv, exb, s_local,
               acc, semE, semA, semB, semS)
    plsc.subcore_barrier()
    _write_stripe(acc, out_hbm.at[cid], rbase)
    pltpu.sync_copy(s_local, sout_hbm.at[wid])
    plsc.subcore_barrier()


def kernel(x, edge_index, conv_w, conv_b, W_l1, b_l1, W_r1, b_r1, att1, bias1,
           W_l2, b_l2, W_r2, b_r2, att2, bias2):
    xflat = jnp.transpose(x, (0, 2, 1)).reshape(N, 48)
    xpad = jnp.zeros((NP, 48), jnp.float32).at[:N].set(xflat)
    wc = conv_w[:, 0, :]
    wbig = jnp.zeros((48, 128), jnp.float32)
    wbig = wbig.at[0:24, 0::2].set(wc.T)
    wbig = wbig.at[24:48, 1::2].set(wc.T)
    cbvec = jnp.repeat(conv_b, 2).reshape(1, 128)

    xl1, xr1 = _proj1(xpad, wbig, cbvec, W_l1, b_l1.reshape(1, 512),
                      W_r1, b_r1.reshape(1, 512))

    loop = jnp.arange(N, dtype=jnp.int32)
    padi = jnp.full((E_PAD - E_RAW - N,), N, jnp.int32)
    src = jnp.concatenate([edge_index[0].astype(jnp.int32), loop, padi])
    dst = jnp.concatenate([edge_index[1].astype(jnp.int32), loop, padi])
    ed = jnp.concatenate([src.reshape(-1, B), dst.reshape(-1, B)], axis=1)
    ed = jnp.concatenate([ed, jnp.full((2, 2 * B), N, jnp.int32)], axis=0)

    tabs, s1 = _sc_gat1(xl1.reshape(NP * 4, 128), xr1.reshape(NP * 4, 128),
                        ed, att1)

    xl2, xr2 = _proj2(tabs, s1, bias1.reshape(1, 512), W_l2,
                      b_l2.reshape(1, 128), W_r2, b_r2.reshape(1, 128))

    parts, s2 = _sc_gat2(xl2, xr2, ed, att2)

    out = _fin(parts, s2, bias2.reshape(1, 128))
    return out[:N]


# speedup vs baseline: 18.4100x; 1.0120x over previous
import functools

import jax
import jax.numpy as jnp
from jax import lax
from jax.experimental import pallas as pl
from jax.experimental.pallas import tpu as pltpu
from jax.experimental.pallas import tpu_sc as plsc

N = 10000
NP = 10240
E_RAW = 320000
E_PAD = 331776
B = 48
BN = 1024


def _proj1_body(x48, wbig, cb, wl, bl, wr, br, xl, xr):
    xf = jnp.dot(x48[...], wbig[...], preferred_element_type=jnp.float32, precision=lax.Precision.HIGHEST)
    xf = xf + cb[...]
    xl[...] = jnp.dot(xf, wl[...], preferred_element_type=jnp.float32, precision=lax.Precision.HIGHEST) + bl[...]
    xr[...] = jnp.dot(xf, wr[...], preferred_element_type=jnp.float32, precision=lax.Precision.HIGHEST) + br[...]


_proj1 = pl.pallas_call(
    _proj1_body,
    grid=(NP // BN,),
    in_specs=[
        pl.BlockSpec((BN, 48), lambda i: (i, 0)),
        pl.BlockSpec((48, 128), lambda i: (0, 0)),
        pl.BlockSpec((1, 128), lambda i: (0, 0)),
        pl.BlockSpec((128, 512), lambda i: (0, 0)),
        pl.BlockSpec((1, 512), lambda i: (0, 0)),
        pl.BlockSpec((128, 512), lambda i: (0, 0)),
        pl.BlockSpec((1, 512), lambda i: (0, 0)),
    ],
    out_specs=[
        pl.BlockSpec((BN, 512), lambda i: (i, 0)),
        pl.BlockSpec((BN, 512), lambda i: (i, 0)),
    ],
    out_shape=[jax.ShapeDtypeStruct((NP, 512), jnp.float32)] * 2,
)


def _proj2_body(tabs, s1, b1, wl, bl, wr, br, xl, xr):
    t = tabs[...]
    ssum = jnp.sum(s1[...], axis=1)
    hs = [t[h] / (ssum[h][:, None] + 1e-16) for h in range(4)]
    h1 = jnp.maximum(jnp.concatenate(hs, axis=1) + b1[...], 0.0)
    xl[...] = jnp.dot(h1, wl[...], preferred_element_type=jnp.float32, precision=lax.Precision.HIGHEST) + bl[...]
    xr[...] = jnp.dot(h1, wr[...], preferred_element_type=jnp.float32, precision=lax.Precision.HIGHEST) + br[...]


_proj2 = pl.pallas_call(
    _proj2_body,
    grid=(NP // BN,),
    in_specs=[
        pl.BlockSpec((4, BN, 128), lambda i: (0, i, 0)),
        pl.BlockSpec((4, 16, BN), lambda i: (0, 0, i)),
        pl.BlockSpec((1, 512), lambda i: (0, 0)),
        pl.BlockSpec((512, 128), lambda i: (0, 0)),
        pl.BlockSpec((1, 128), lambda i: (0, 0)),
        pl.BlockSpec((512, 128), lambda i: (0, 0)),
        pl.BlockSpec((1, 128), lambda i: (0, 0)),
    ],
    out_specs=[
        pl.BlockSpec((BN, 128), lambda i: (i, 0)),
        pl.BlockSpec((BN, 128), lambda i: (i, 0)),
    ],
    out_shape=[jax.ShapeDtypeStruct((NP, 128), jnp.float32)] * 2,
)


def _fin_body(p, s2, b2, o):
    t = p[...]
    den = jnp.sum(s2[...], axis=0)[:, None] + 1e-16
    o[...] = (t[0] + t[1]) / den + b2[...]


_fin = pl.pallas_call(
    _fin_body,
    grid=(NP // BN,),
    in_specs=[
        pl.BlockSpec((2, BN, 128), lambda i: (0, i, 0)),
        pl.BlockSpec((32, BN), lambda i: (0, i)),
        pl.BlockSpec((1, 128), lambda i: (0, 0)),
    ],
    out_specs=pl.BlockSpec((BN, 128), lambda i: (i, 0)),
    out_shape=jax.ShapeDtypeStruct((NP, 128), jnp.float32),
)


_MESH = plsc.VectorSubcoreMesh(core_axis_name="c", subcore_axis_name="s",
                               num_cores=2, num_subcores=16)
_SC_PARAMS = pltpu.CompilerParams(needs_layout_passes=False,
                                 internal_scratch_in_bytes=65536)


def _idx_phase(h_or_none, ebuf, row, gia, gib, gio, dstw):
    for k in range(B // 16):
        s16 = ebuf[row, pl.ds(16 * k, 16)]
        d16 = ebuf[row, pl.ds(B + 16 * k, 16)]
        if h_or_none is None:
            gia[pl.ds(16 * k, 16)] = s16
            gib[pl.ds(16 * k, 16)] = d16
        else:
            gia[pl.ds(16 * k, 16)] = s16 * 4 + h_or_none
            gib[pl.ds(16 * k, 16)] = d16 * 4 + h_or_none
        gio[pl.ds(16 * k, 16)] = d16
        dstw[pl.ds(16 * k, 16)] = d16


def _compute_chunk(att06, gio, dstw, Av, Bv, Mv, exb, s_local, acc,
                   semS):
    lanes = lax.iota(jnp.int32, 16)

    @plsc.parallel_loop(0, B, unroll=1)
    def _score(e):
        acc1 = None
        acc2 = None
        for j in range(8):
            a = Av[e, pl.ds(16 * j, 16)]
            t = a + Bv[e, pl.ds(16 * j, 16)]
            u1 = t * att06[j]
            u2 = jnp.abs(t) * att06[j]
            acc1 = u1 if acc1 is None else acc1 + u1
            acc2 = u2 if acc2 is None else acc2 + u2
        score = jnp.sum(acc1) + (2.0 / 3.0) * jnp.sum(acc2)
        exv = jnp.exp(jnp.zeros((16,), jnp.float32) + score)
        for j in range(8):
            Mv[e, pl.ds(16 * j, 16)] = Av[e, pl.ds(16 * j, 16)] * exv
        exb[e, pl.ds(0, 16)] = exv

    cps = pltpu.async_copy(Mv, acc.at[gio], semS, add=True)

    @pl.loop(0, B)
    def _spass(e):
        d = dstw[pl.ds(e, 16)][0]
        q = (d >> 4) << 4
        ln = d & 15
        oh = (lanes == ln).astype(jnp.float32)
        exv = exb[e, pl.ds(0, 16)]
        s_local[pl.ds(q, 16)] = s_local[pl.ds(q, 16)] + oh * exv

    cps.wait()


def _edge_pass(h_or_none, nch, tbase, att06, ed_hbm, xl_hbm, xr_hbm,
               ebuf, gia, gib, gio, dstw, Av, Bv, Mv, exb, s_local, acc,
               semE, semA, semB, semS):
    def issue(sl):
        pltpu.async_copy(xl_hbm.at[gia.at[sl]], Av.at[sl], semA.at[sl])
        pltpu.async_copy(xr_hbm.at[gib.at[sl]], Bv.at[sl], semB.at[sl])

    def wait(sl):
        pltpu.make_async_copy(xl_hbm.at[gia.at[sl]], Av.at[sl],
                              semA.at[sl]).wait()
        pltpu.make_async_copy(xr_hbm.at[gib.at[sl]], Bv.at[sl],
                              semB.at[sl]).wait()

    def compute(sl):
        _compute_chunk(att06, gio.at[sl], dstw.at[sl], Av.at[sl],
                       Bv.at[sl], Mv, exb, s_local, acc, semS)

    pltpu.sync_copy(ed_hbm.at[pl.ds(tbase, 2)], ebuf.at[0])
    for sl in range(2):
        _idx_phase(h_or_none, ebuf.at[0], sl, gia.at[sl], gib.at[sl],
                   gio.at[sl], dstw.at[sl])
        issue(sl)

    @pl.loop(0, nch, step=2)
    def _body(g):
        p = (g // 2) & 1
        pltpu.async_copy(ed_hbm.at[pl.ds(tbase + g + 2, 2)], ebuf.at[1 - p],
                         semE)
        wait(0)
        compute(0)
        pltpu.make_async_copy(ed_hbm.at[pl.ds(tbase + g + 2, 2)],
                              ebuf.at[1 - p], semE).wait()
        _idx_phase(h_or_none, ebuf.at[1 - p], 0, gia.at[0], gib.at[0],
                   gio.at[0], dstw.at[0])
        issue(0)
        wait(1)
        compute(1)
        _idx_phase(h_or_none, ebuf.at[1 - p], 1, gia.at[1], gib.at[1],
                   gio.at[1], dstw.at[1])
        issue(1)

    wait(0)
    wait(1)


def _zero_slocal(s_local):
    z16 = jnp.zeros((16,), jnp.float32)

    @pl.loop(0, NP // 16)
    def _(i):
        s_local[pl.ds(i * 16, 16)] = z16


def _zero_rows(ref):
    z16 = jnp.zeros((16,), jnp.float32)

    @pl.loop(0, ref.shape[0])
    def _(r):
        for j in range(ref.shape[1] // 16):
            ref[r, pl.ds(16 * j, 16)] = z16


def _zero_stripe(Mv, dst, rbase):
    nfull = (NP // 16) // B * B
    for k in range((NP // 16) // B):
        pltpu.sync_copy(Mv, dst.at[pl.ds(rbase + B * k, B)])
    rem = NP // 16 - nfull
    if rem:
        pltpu.sync_copy(Mv.at[pl.ds(0, rem)],
                        dst.at[pl.ds(rbase + nfull, rem)])


def _write_stripe(acc, out, rbase):
    nfull = (NP // 16) // B * B
    for k in range((NP // 16) // B):
        sl = pl.ds(rbase + B * k, B)
        pltpu.sync_copy(acc.at[sl], out.at[sl])
    rem = NP // 16 - nfull
    if rem:
        sl = pl.ds(rbase + nfull, rem)
        pltpu.sync_copy(acc.at[sl], out.at[sl])


_NCH1 = E_PAD // 16 // B


@functools.partial(
    pl.kernel,
    out_type=[jax.ShapeDtypeStruct((4, NP, 128), jnp.float32),
              jax.ShapeDtypeStruct((4, 16, NP), jnp.float32)],
    mesh=_MESH,
    compiler_params=_SC_PARAMS,
    scratch_types=[
        pltpu.VMEM((2, 2, 2 * B), jnp.int32),
        pltpu.VMEM((4, 128), jnp.float32),
        pltpu.VMEM((2, B), jnp.int32),
        pltpu.VMEM((2, B), jnp.int32),
        pltpu.VMEM((2, B), jnp.int32),
        pltpu.VMEM((2, B + 16), jnp.int32),
        pltpu.VMEM((2, B, 128), jnp.float32),
        pltpu.VMEM((2, B, 128), jnp.float32),
        pltpu.VMEM((B, 128), jnp.float32),
        pltpu.VMEM((B, 16), jnp.float32),
        pltpu.VMEM((NP,), jnp.float32),
        pltpu.VMEM_SHARED((NP, 128), jnp.float32),
        pltpu.SemaphoreType.DMA,
        pltpu.SemaphoreType.DMA((2,)),
        pltpu.SemaphoreType.DMA((2,)),
        pltpu.SemaphoreType.DMA,
    ],
)
def _sc_gat1(xl_hbm, xr_hbm, ed_hbm, att_hbm, out_hbm, sout_hbm,
             ebuf, attv, gia, gib, gio, dstw, Av, Bv, Mv, exb, s_local,
             acc, semE, semA, semB, semS):
    cid = lax.axis_index("c")
    sid = lax.axis_index("s")
    pltpu.sync_copy(att_hbm, attv)
    rbase = sid * (NP // 16)
    for hh in range(2):
        h = cid * 2 + hh
        att06 = [attv[h, pl.ds(16 * j, 16)] * 0.6 for j in range(8)]
        _zero_slocal(s_local)
        _zero_rows(Mv)
        _zero_stripe(Mv, acc, rbase)
        plsc.subcore_barrier()
        _edge_pass(h, _NCH1, sid * _NCH1, att06, ed_hbm, xl_hbm,
                   xr_hbm, ebuf, gia, gib, gio, dstw, Av, Bv, Mv, exb,
                   s_local, acc, semE, semA, semB, semS)
        plsc.subcore_barrier()
        _write_stripe(acc, out_hbm.at[h], rbase)
        pltpu.sync_copy(s_local, sout_hbm.at[h, sid])
        plsc.subcore_barrier()


_NCH2 = E_PAD // 32 // B


@functools.partial(
    pl.kernel,
    out_type=[jax.ShapeDtypeStruct((2, NP, 128), jnp.float32),
              jax.ShapeDtypeStruct((32, NP), jnp.float32)],
    mesh=_MESH,
    compiler_params=_SC_PARAMS,
    scratch_types=[
        pltpu.VMEM((2, 2, 2 * B), jnp.int32),
        pltpu.VMEM((1, 128), jnp.float32),
        pltpu.VMEM((2, B), jnp.int32),
        pltpu.VMEM((2, B), jnp.int32),
        pltpu.VMEM((2, B), jnp.int32),
        pltpu.VMEM((2, B + 16), jnp.int32),
        pltpu.VMEM((2, B, 128), jnp.float32),
        pltpu.VMEM((2, B, 128), jnp.float32),
        pltpu.VMEM((B, 128), jnp.float32),
        pltpu.VMEM((B, 16), jnp.float32),
        pltpu.VMEM((NP,), jnp.float32),
        pltpu.VMEM_SHARED((NP, 128), jnp.float32),
        pltpu.SemaphoreType.DMA,
        pltpu.SemaphoreType.DMA((2,)),
        pltpu.SemaphoreType.DMA((2,)),
        pltpu.SemaphoreType.DMA,
    ],
)
def _sc_gat2(xl_hbm, xr_hbm, ed_hbm, att_hbm, out_hbm, sout_hbm,
             ebuf, attv, gia, gib, gio, dstw, Av, Bv, Mv, exb, s_local,
             acc, semE, semA, semB, semS):
    cid = lax.axis_index("c")
    sid = lax.axis_index("s")
    wid = sid * 2 + cid
    pltpu.sync_copy(att_hbm, attv)
    rbase = sid * (NP // 16)
    att06 = [attv[0, pl.ds(16 * j, 16)] * 0.6 for j in range(8)]
    _zero_slocal(s_local)
    _zero_rows(Mv)
    _zero_stripe(Mv, acc, rbase)
    plsc.subcore_barrier()
    _edge_pass(None, _NCH2, wid * _NCH2, att06, ed_hbm, xl_hbm,
               xr_hbm, ebuf, gia, gib, gio, dstw, Av, Bv, Mv, exb, s_local,
               acc, semE, semA, semB, semS)
    plsc.subcore_barrier()
    _write_stripe(acc, out_hbm.at[cid], rbase)
    pltpu.sync_copy(s_local, sout_hbm.at[wid])
    plsc.subcore_barrier()


def kernel(x, edge_index, conv_w, conv_b, W_l1, b_l1, W_r1, b_r1, att1, bias1,
           W_l2, b_l2, W_r2, b_r2, att2, bias2):
    xflat = jnp.transpose(x, (0, 2, 1)).reshape(N, 48)
    xpad = jnp.zeros((NP, 48), jnp.float32).at[:N].set(xflat)
    wc = conv_w[:, 0, :]
    wbig = jnp.zeros((48, 128), jnp.float32)
    wbig = wbig.at[0:24, 0::2].set(wc.T)
    wbig = wbig.at[24:48, 1::2].set(wc.T)
    cbvec = jnp.repeat(conv_b, 2).reshape(1, 128)

    xl1, xr1 = _proj1(xpad, wbig, cbvec, W_l1, b_l1.reshape(1, 512),
                      W_r1, b_r1.reshape(1, 512))

    loop = jnp.arange(N, dtype=jnp.int32)
    padi = jnp.full((E_PAD - E_RAW - N,), N, jnp.int32)
    src = jnp.concatenate([edge_index[0].astype(jnp.int32), loop, padi])
    dst = jnp.concatenate([edge_index[1].astype(jnp.int32), loop, padi])
    ed = jnp.concatenate([src.reshape(-1, B), dst.reshape(-1, B)], axis=1)
    ed = jnp.concatenate([ed, jnp.full((2, 2 * B), N, jnp.int32)], axis=0)

    tabs, s1 = _sc_gat1(xl1.reshape(NP * 4, 128), xr1.reshape(NP * 4, 128),
                        ed, att1)

    xl2, xr2 = _proj2(tabs, s1, bias1.reshape(1, 512), W_l2,
                      b_l2.reshape(1, 128), W_r2, b_r2.reshape(1, 128))

    parts, s2 = _sc_gat2(xl2, xr2, ed, att2)

    out = _fin(parts, s2, bias2.reshape(1, 128))
    return out[:N]
